# baseline (speedup 1.0000x reference)
import functools

import jax
import jax.numpy as jnp
from jax import lax
from jax.experimental import pallas as pl
from jax.experimental.pallas import tpu as pltpu
from jax.experimental.pallas import tpu_sc as plsc

N_FINE = 50000
N_GLOBAL = 500
E = 500000
IN_F = 128
HID = 256
ATT = 16
HEADS = 8
DG = ATT * HEADS
LLM = 768

NW = 32
CT = 15680
CHUNKS = CT // 16
NBUF = 4
EPAD = NW * CT
TW = 144

_BF = 5000


def _pre_body(x_ref, gf_ref, bf_ref, w1_ref, b1_ref, w2_ref, b2_ref,
              ws_ref, a_ref, t_ref, as_ref, xf_ref, mx_ref):
    i = pl.program_id(0)
    x = x_ref[...]
    m = jnp.mean(x, axis=-1, keepdims=True)
    v = jnp.mean((x - m) ** 2, axis=-1, keepdims=True)
    xln = (x - m) * lax.rsqrt(v + 1e-5) * gf_ref[...] + bf_ref[...]
    bf16 = jnp.bfloat16
    h1 = jnp.maximum(
        jnp.dot(xln.astype(bf16), w1_ref[...], preferred_element_type=jnp.float32)
        + b1_ref[...], 0.0)
    xf = (jnp.dot(h1.astype(bf16), w2_ref[...], preferred_element_type=jnp.float32)
          + b2_ref[...])
    hs = jnp.dot(xf.astype(bf16), ws_ref[...], preferred_element_type=jnp.float32)
    asrc = jnp.dot(hs, a_ref[...], preferred_element_type=jnp.float32)
    xf_ref[...] = xf
    t_ref[...] = hs
    as_ref[...] = asrc
    bmax = jnp.max(asrc, axis=0, keepdims=True)

    @pl.when(i == 0)
    def _():
        mx_ref[...] = bmax

    @pl.when(i > 0)
    def _():
        mx_ref[...] = jnp.maximum(mx_ref[...], bmax)


def _run_pre(x_fine, gf, bf, W1, b1, W2, b2, W_src, A):
    nblk = N_FINE // _BF
    return pl.pallas_call(
        _pre_body,
        grid=(nblk,),
        in_specs=[
            pl.BlockSpec((_BF, IN_F), lambda i: (i, 0)),
            pl.BlockSpec((1, IN_F), lambda i: (0, 0)),
            pl.BlockSpec((1, IN_F), lambda i: (0, 0)),
            pl.BlockSpec((IN_F, HID), lambda i: (0, 0)),
            pl.BlockSpec((1, HID), lambda i: (0, 0)),
            pl.BlockSpec((HID, DG), lambda i: (0, 0)),
            pl.BlockSpec((1, DG), lambda i: (0, 0)),
            pl.BlockSpec((DG, DG), lambda i: (0, 0)),
            pl.BlockSpec((DG, HEADS), lambda i: (0, 0)),
        ],
        out_specs=[
            pl.BlockSpec((_BF, DG), lambda i: (i, 0)),
            pl.BlockSpec((_BF, HEADS), lambda i: (i, 0)),
            pl.BlockSpec((_BF, IN_F), lambda i: (i, 0)),
            pl.BlockSpec((1, HEADS), lambda i: (0, 0)),
        ],
        out_shape=[
            jax.ShapeDtypeStruct((N_FINE, DG), jnp.float32),
            jax.ShapeDtypeStruct((N_FINE, HEADS), jnp.float32),
            jax.ShapeDtypeStruct((N_FINE, IN_F), jnp.float32),
            jax.ShapeDtypeStruct((1, HEADS), jnp.float32),
        ],
    )(x_fine, gf, bf, W1, b1, W2, b2, W_src, A)


def _glob_body(xg_ref, gg_ref, bg_ref, wd_ref, ad_ref, mx_ref,
               xgln_ref, adstb_ref):
    xg = xg_ref[...]
    m = jnp.mean(xg, axis=-1, keepdims=True)
    v = jnp.mean((xg - m) ** 2, axis=-1, keepdims=True)
    xgln = (xg - m) * lax.rsqrt(v + 1e-5) * gg_ref[...] + bg_ref[...]
    hd = jnp.dot(xgln, wd_ref[...], preferred_element_type=jnp.float32)
    adst = jnp.dot(hd, ad_ref[...], preferred_element_type=jnp.float32)
    s = mx_ref[...] + adst
    bub = jnp.maximum(s, 0.2 * s)
    xgln_ref[...] = xgln
    adstb_ref[...] = jnp.concatenate([adst, bub], axis=1)


def _run_glob(x_global, gg, bg, W_dst, Ad, mx):
    return pl.pallas_call(
        _glob_body,
        out_shape=[
            jax.ShapeDtypeStruct((N_GLOBAL, DG), jnp.float32),
            jax.ShapeDtypeStruct((N_GLOBAL, 16), jnp.float32),
        ],
    )(x_global, gg, bg, W_dst, Ad, mx)


_GDN = lax.GatherDimensionNumbers(
    offset_dims=(), collapsed_slice_dims=(0,), start_index_map=(0,))


def _lane_splat(vec, lane_idx):
    return lax.gather(vec, lane_idx[:, None], _GDN, slice_sizes=(1,),
                      mode=lax.GatherScatterMode.PROMISE_IN_BOUNDS)


def _edge_body(src_hbm, dst_hbm, t_hbm, as8_hbm, adstb_hbm, out_hbm,
               src_v, dst_v, adstb_v, agg_v, rows_v, idx_v, aidx_v, asb_v,
               exem_v, dsem, asem):
    c = lax.axis_index("c")
    s = lax.axis_index("s")
    wid = s * 2 + c
    base0 = wid * CT

    pltpu.sync_copy(src_hbm.at[pl.ds(base0, CT)], src_v)
    pltpu.sync_copy(dst_hbm.at[pl.ds(base0, CT)], dst_v.at[pl.ds(0, CT)])
    pltpu.sync_copy(adstb_hbm, adstb_v)

    zz = jnp.zeros((16,), jnp.float32)

    def _zero(i, carry):
        agg_v[pl.ds(i * 16, 16)] = zz
        return carry

    lax.fori_loop(0, (N_GLOBAL * TW) // 16, _zero, 0)
    for e in range(16):
        exem_v[e, :] = zz

    iota16 = lax.iota(jnp.int32, 16)

    def _prefetch(b, ch):
        srcn = src_v[pl.ds(ch * 16, 16)]
        idx_v[b, :] = srcn
        srcn8 = srcn * 8
        for h in range(HEADS):
            aidx_v[b, pl.ds(h * 16, 16)] = srcn8 + h
        pltpu.make_async_copy(t_hbm.at[idx_v.at[b]], rows_v.at[b],
                              dsem.at[b]).start()
        pltpu.make_async_copy(as8_hbm.at[aidx_v.at[b]], asb_v.at[b],
                              asem.at[b]).start()

    for b in range(NBUF):
        _prefetch(b, b)

    def _chunk_group(g, carry):
        for b in range(NBUF):
            ch = g * NBUF + b
            pltpu.make_async_copy(t_hbm.at[idx_v.at[b]], rows_v.at[b],
                                  dsem.at[b]).wait()
            pltpu.make_async_copy(as8_hbm.at[aidx_v.at[b]], asb_v.at[b],
                                  asem.at[b]).wait()
            rb = rows_v.at[b]
            dstv = dst_v[pl.ds(ch * 16, 16)]
            eid = base0 + ch * 16 + iota16
            emask = eid < E
            dstv16 = dstv * 16

            @plsc.parallel_loop(0, HEADS, 1, unroll=HEADS)
            def _head(h):
                a_s = asb_v[b, pl.ds(h * 16, 16)]
                a_d = plsc.load_gather(adstb_v, [dstv16 + h])
                bub = plsc.load_gather(adstb_v, [dstv16 + (HEADS + h)])
                sm = a_s + a_d
                al = jnp.maximum(sm, 0.2 * sm)
                exh = jnp.where(emask, jnp.exp(al - bub), 0.0)
                plsc.store_scatter(
                    exem_v, [iota16, jnp.full((16,), h, jnp.int32)], exh)

            @plsc.parallel_loop(0, 16, 1, unroll=16)
            def _acc(e):
                dvec = dst_v[pl.ds(ch * 16 + e, 16)]
                dbase = dvec[0] * TW
                exrow = exem_v[e, :]
                plsc.addupdate(agg_v.at[pl.ds(dbase + DG, 16)], exrow)
                for h in range(HEADS):
                    scal = _lane_splat(exrow, jnp.full((16,), h, jnp.int32))
                    seg = rb[e, pl.ds(h * 16, 16)]
                    plsc.addupdate(agg_v.at[pl.ds(dbase + h * 16, 16)],
                                   seg * scal)

            nxt = ch + NBUF

            @pl.when(nxt < CHUNKS)
            def _():
                _prefetch(b, nxt)
        return carry

    lax.fori_loop(0, CHUNKS // NBUF, _chunk_group, 0)
    pltpu.sync_copy(agg_v, out_hbm.at[pl.ds(wid * (N_GLOBAL * TW),
                                            N_GLOBAL * TW)])


def _run_edges(src_pad, dst_pad, T, as8, adstb):
    mesh = plsc.VectorSubcoreMesh(core_axis_name="c", subcore_axis_name="s")
    fn = functools.partial(
        pl.kernel,
        mesh=mesh,
        compiler_params=pltpu.CompilerParams(
            needs_layout_passes=False, use_tc_tiling_on_sc=True),
        out_type=jax.ShapeDtypeStruct((NW * N_GLOBAL * TW,), jnp.float32),
        scratch_types=[
            pltpu.VMEM((CT,), jnp.int32),
            pltpu.VMEM((CT + 16,), jnp.int32),
            pltpu.VMEM((N_GLOBAL * 16,), jnp.float32),
            pltpu.VMEM((N_GLOBAL * TW,), jnp.float32),
            pltpu.VMEM((NBUF, 16, DG), jnp.float32),
            pltpu.VMEM((NBUF, 16), jnp.int32),
            pltpu.VMEM((NBUF, 128), jnp.int32),
            pltpu.VMEM((NBUF, 128), jnp.float32),
            pltpu.VMEM((16, 16), jnp.float32),
            pltpu.SemaphoreType.DMA((NBUF,)),
            pltpu.SemaphoreType.DMA((NBUF,)),
        ],
    )(_edge_body)
    return fn(src_pad, dst_pad, T, as8, adstb)


def _out_body(xf3_ref, wf_ref, bfp_ref, parts_ref, tg_ref, asl_ref, adstb_ref,
              xgln_ref, bias_ref, r_ref, wg_ref, bgp_ref, out_ref):
    j = pl.program_id(0)

    @pl.when(j == 0)
    def _():
        p = jnp.sum(parts_ref[...], axis=0)
        snum = p[:, :DG]
        den = p[:, DG:DG + HEADS]
        tg = tg_ref[...]
        asl = asl_ref[...]
        ad = adstb_ref[:, :HEADS]
        bub = adstb_ref[:, HEADS:]
        sm = asl + ad
        al = jnp.maximum(sm, 0.2 * sm)
        exl = jnp.exp(al - bub)
        rmat = r_ref[...]
        exrep = jnp.dot(exl, rmat, preferred_element_type=jnp.float32)
        snum = snum + exrep * tg
        drep = (jnp.dot(den + exl, rmat, preferred_element_type=jnp.float32)
                + 1e-16)
        xgn = snum / drep + bias_ref[...] + xgln_ref[...]
        out_ref[...] = (
            jnp.dot(xgn.astype(jnp.bfloat16), wg_ref[...],
                    preferred_element_type=jnp.float32)
            + bgp_ref[...]).reshape(1, N_GLOBAL, LLM)

    @pl.when(j > 0)
    def _():
        xfb = xf3_ref[...].reshape(N_GLOBAL, IN_F)
        out_ref[...] = (
            jnp.dot(xfb.astype(jnp.bfloat16), wf_ref[...],
                    preferred_element_type=jnp.float32)
            + bfp_ref[...]).reshape(1, N_GLOBAL, LLM)


def _run_out(xf3, Wf, bf, parts, Tg, asl, adstb, xgln, bias, R, Wg, bg):
    nj = N_FINE // N_GLOBAL + 1
    return pl.pallas_call(
        _out_body,
        grid=(nj,),
        in_specs=[
            pl.BlockSpec((N_GLOBAL, 1, 1, IN_F),
                         lambda j: (0, jnp.maximum(j - 1, 0), 0, 0)),
            pl.BlockSpec((IN_F, LLM), lambda j: (0, 0)),
            pl.BlockSpec((1, LLM), lambda j: (0, 0)),
            pl.BlockSpec((NW, N_GLOBAL, TW), lambda j: (0, 0, 0)),
            pl.BlockSpec((N_GLOBAL, DG), lambda j: (0, 0)),
            pl.BlockSpec((N_GLOBAL, HEADS), lambda j: (0, 0)),
            pl.BlockSpec((N_GLOBAL, 16), lambda j: (0, 0)),
            pl.BlockSpec((N_GLOBAL, DG), lambda j: (0, 0)),
            pl.BlockSpec((1, DG), lambda j: (0, 0)),
            pl.BlockSpec((HEADS, DG), lambda j: (0, 0)),
            pl.BlockSpec((IN_F, LLM), lambda j: (0, 0)),
            pl.BlockSpec((1, LLM), lambda j: (0, 0)),
        ],
        out_specs=pl.BlockSpec((1, N_GLOBAL, LLM), lambda j: (j, 0, 0)),
        out_shape=jax.ShapeDtypeStruct((nj, N_GLOBAL, LLM), jnp.float32),
    )(xf3, Wf, bf, parts, Tg, asl, adstb, xgln, bias, R, Wg, bg)


def kernel(x_fine, x_global, edge_index, gamma_f, beta_f, W1, b1, W2, b2,
           gamma_g, beta_g, W_src, W_dst, att_src, att_dst, bias_gat,
           Wg, bg, Wf, bf):
    f32 = jnp.float32
    eye = jnp.eye(HEADS, dtype=f32)
    A = (att_src[:, :, None] * eye[:, None, :]).reshape(DG, HEADS)
    Ad = (att_dst[:, :, None] * eye[:, None, :]).reshape(DG, HEADS)
    R = jnp.broadcast_to(eye[:, :, None], (HEADS, HEADS, ATT)).reshape(HEADS, DG)

    src = edge_index[0]
    dst = edge_index[1]
    padn = EPAD - E
    src_pad = jnp.concatenate([src, jnp.zeros((padn,), src.dtype)])
    dst_pad = jnp.concatenate([dst, jnp.zeros((padn,), dst.dtype)])

    bf16 = jnp.bfloat16
    T, asrc, xf, mx = _run_pre(
        x_fine, gamma_f.reshape(1, IN_F), beta_f.reshape(1, IN_F),
        W1.astype(bf16), b1.reshape(1, HID), W2.astype(bf16),
        b2.reshape(1, DG), W_src.astype(bf16), A)
    xgln, adstb = _run_glob(
        x_global, gamma_g.reshape(1, DG), beta_g.reshape(1, DG), W_dst, Ad, mx)
    parts = _run_edges(src_pad, dst_pad, T, asrc.reshape(N_FINE * HEADS),
                       adstb.reshape(N_GLOBAL * 16))
    parts = parts.reshape(NW, N_GLOBAL, TW)
    Tg = lax.slice(T, (0, 0), (N_GLOBAL, DG))
    asl = lax.slice(asrc, (0, 0), (N_GLOBAL, HEADS))
    xf3 = xf.reshape(N_GLOBAL, N_FINE // N_GLOBAL, 1, IN_F)
    full = _run_out(xf3, Wf.astype(bf16), bf.reshape(1, LLM), parts, Tg, asl,
                    adstb, xgln, bias_gat.reshape(1, DG), R, Wg.astype(bf16),
                    bg.reshape(1, LLM))
    return jnp.swapaxes(full, 0, 1)


# speedup vs baseline: 1.1705x; 1.1705x over previous
import functools

import jax
import jax.numpy as jnp
from jax import lax
from jax.experimental import pallas as pl
from jax.experimental.pallas import tpu as pltpu
from jax.experimental.pallas import tpu_sc as plsc

N_FINE = 50000
N_GLOBAL = 500
E = 500000
IN_F = 128
HID = 256
ATT = 16
HEADS = 8
DG = ATT * HEADS
LLM = 768

NW = 32
CT = 15680
CHUNKS = CT // 16
NBUF = 4
EPAD = NW * CT
TW = 144

_BF = 5000


def _pre_body(x_ref, gf_ref, bf_ref, w1_ref, b1_ref, w2_ref, b2_ref,
              ws_ref, a_ref, t_ref, as_ref, xf_ref, mx_ref):
    i = pl.program_id(0)
    x = x_ref[...]
    m = jnp.mean(x, axis=-1, keepdims=True)
    v = jnp.mean((x - m) ** 2, axis=-1, keepdims=True)
    xln = (x - m) * lax.rsqrt(v + 1e-5) * gf_ref[...] + bf_ref[...]
    h1 = jnp.maximum(
        jnp.dot(xln, w1_ref[...], preferred_element_type=jnp.float32)
        + b1_ref[...], 0.0)
    xf = jnp.dot(h1, w2_ref[...], preferred_element_type=jnp.float32) + b2_ref[...]
    hs = jnp.dot(xf, ws_ref[...], preferred_element_type=jnp.float32)
    asrc = jnp.dot(hs, a_ref[...], preferred_element_type=jnp.float32)
    xf_ref[...] = xf
    t_ref[...] = hs
    as_ref[...] = asrc
    bmax = jnp.max(asrc, axis=0, keepdims=True)

    @pl.when(i == 0)
    def _():
        mx_ref[...] = bmax

    @pl.when(i > 0)
    def _():
        mx_ref[...] = jnp.maximum(mx_ref[...], bmax)


def _run_pre(x_fine, gf, bf, W1, b1, W2, b2, W_src, A):
    nblk = N_FINE // _BF
    return pl.pallas_call(
        _pre_body,
        grid=(nblk,),
        in_specs=[
            pl.BlockSpec((_BF, IN_F), lambda i: (i, 0)),
            pl.BlockSpec((1, IN_F), lambda i: (0, 0)),
            pl.BlockSpec((1, IN_F), lambda i: (0, 0)),
            pl.BlockSpec((IN_F, HID), lambda i: (0, 0)),
            pl.BlockSpec((1, HID), lambda i: (0, 0)),
            pl.BlockSpec((HID, DG), lambda i: (0, 0)),
            pl.BlockSpec((1, DG), lambda i: (0, 0)),
            pl.BlockSpec((DG, DG), lambda i: (0, 0)),
            pl.BlockSpec((DG, HEADS), lambda i: (0, 0)),
        ],
        out_specs=[
            pl.BlockSpec((_BF, DG), lambda i: (i, 0)),
            pl.BlockSpec((_BF, HEADS), lambda i: (i, 0)),
            pl.BlockSpec((_BF, IN_F), lambda i: (i, 0)),
            pl.BlockSpec((1, HEADS), lambda i: (0, 0)),
        ],
        out_shape=[
            jax.ShapeDtypeStruct((N_FINE, DG), jnp.float32),
            jax.ShapeDtypeStruct((N_FINE, HEADS), jnp.float32),
            jax.ShapeDtypeStruct((N_FINE, IN_F), jnp.float32),
            jax.ShapeDtypeStruct((1, HEADS), jnp.float32),
        ],
    )(x_fine, gf, bf, W1, b1, W2, b2, W_src, A)


def _glob_body(xg_ref, gg_ref, bg_ref, wd_ref, ad_ref, mx_ref,
               xgln_ref, adstb_ref):
    xg = xg_ref[...]
    m = jnp.mean(xg, axis=-1, keepdims=True)
    v = jnp.mean((xg - m) ** 2, axis=-1, keepdims=True)
    xgln = (xg - m) * lax.rsqrt(v + 1e-5) * gg_ref[...] + bg_ref[...]
    hd = jnp.dot(xgln, wd_ref[...], preferred_element_type=jnp.float32)
    adst = jnp.dot(hd, ad_ref[...], preferred_element_type=jnp.float32)
    s = mx_ref[...] + adst
    bub = jnp.maximum(s, 0.2 * s)
    xgln_ref[...] = xgln
    adstb_ref[...] = jnp.concatenate([adst, bub], axis=1)


def _run_glob(x_global, gg, bg, W_dst, Ad, mx):
    return pl.pallas_call(
        _glob_body,
        out_shape=[
            jax.ShapeDtypeStruct((N_GLOBAL, DG), jnp.float32),
            jax.ShapeDtypeStruct((N_GLOBAL, 16), jnp.float32),
        ],
    )(x_global, gg, bg, W_dst, Ad, mx)


_GDN = lax.GatherDimensionNumbers(
    offset_dims=(), collapsed_slice_dims=(0,), start_index_map=(0,))


def _lane_splat(vec, lane_idx):
    return lax.gather(vec, lane_idx[:, None], _GDN, slice_sizes=(1,),
                      mode=lax.GatherScatterMode.PROMISE_IN_BOUNDS)


def _edge_body(src_hbm, dst_hbm, t_hbm, as8_hbm, adstb_hbm, out_hbm,
               src_v, dst_v, adstb_v, agg_v, rows_v, idx_v, aidx_v, asb_v,
               exem_v, dsem, asem):
    c = lax.axis_index("c")
    s = lax.axis_index("s")
    wid = s * 2 + c
    base0 = wid * CT

    pltpu.sync_copy(src_hbm.at[pl.ds(base0, CT)], src_v)
    pltpu.sync_copy(dst_hbm.at[pl.ds(base0, CT)], dst_v.at[pl.ds(0, CT)])
    pltpu.sync_copy(adstb_hbm, adstb_v)

    zz = jnp.zeros((16,), jnp.float32)

    def _zero(i, carry):
        agg_v[pl.ds(i * 16, 16)] = zz
        return carry

    lax.fori_loop(0, (N_GLOBAL * TW) // 16, _zero, 0)
    for e in range(16):
        exem_v[e, :] = zz

    iota16 = lax.iota(jnp.int32, 16)

    def _prefetch(b, ch):
        srcn = src_v[pl.ds(ch * 16, 16)]
        idx_v[b, :] = srcn
        srcn8 = srcn * 8
        for h in range(HEADS):
            aidx_v[b, pl.ds(h * 16, 16)] = srcn8 + h
        pltpu.make_async_copy(t_hbm.at[idx_v.at[b]], rows_v.at[b],
                              dsem.at[b]).start()
        pltpu.make_async_copy(as8_hbm.at[aidx_v.at[b]], asb_v.at[b],
                              asem.at[b]).start()

    for b in range(NBUF):
        _prefetch(b, b)

    def _chunk_group(g, carry):
        for b in range(NBUF):
            ch = g * NBUF + b
            pltpu.make_async_copy(t_hbm.at[idx_v.at[b]], rows_v.at[b],
                                  dsem.at[b]).wait()
            pltpu.make_async_copy(as8_hbm.at[aidx_v.at[b]], asb_v.at[b],
                                  asem.at[b]).wait()
            rb = rows_v.at[b]
            dstv = dst_v[pl.ds(ch * 16, 16)]
            eid = base0 + ch * 16 + iota16
            emask = eid < E
            dstv16 = dstv * 16

            @plsc.parallel_loop(0, HEADS, 1, unroll=HEADS)
            def _head(h):
                a_s = asb_v[b, pl.ds(h * 16, 16)]
                a_d = plsc.load_gather(adstb_v, [dstv16 + h])
                bub = plsc.load_gather(adstb_v, [dstv16 + (HEADS + h)])
                sm = a_s + a_d
                al = jnp.maximum(sm, 0.2 * sm)
                exh = jnp.where(emask, jnp.exp(al - bub), 0.0)
                plsc.store_scatter(
                    exem_v, [iota16, jnp.full((16,), h, jnp.int32)], exh)

            @plsc.parallel_loop(0, 16, 1, unroll=16)
            def _acc(e):
                dvec = dst_v[pl.ds(ch * 16 + e, 16)]
                dbase = dvec[0] * TW
                exrow = exem_v[e, :]
                plsc.addupdate(agg_v.at[pl.ds(dbase + DG, 16)], exrow)
                for h in range(HEADS):
                    scal = _lane_splat(exrow, jnp.full((16,), h, jnp.int32))
                    seg = rb[e, pl.ds(h * 16, 16)]
                    plsc.addupdate(agg_v.at[pl.ds(dbase + h * 16, 16)],
                                   seg * scal)

            nxt = ch + NBUF

            @pl.when(nxt < CHUNKS)
            def _():
                _prefetch(b, nxt)
        return carry

    lax.fori_loop(0, CHUNKS // NBUF, _chunk_group, 0)
    pltpu.sync_copy(agg_v, out_hbm.at[pl.ds(wid * (N_GLOBAL * TW),
                                            N_GLOBAL * TW)])


def _run_edges(src_pad, dst_pad, T, as8, adstb):
    mesh = plsc.VectorSubcoreMesh(core_axis_name="c", subcore_axis_name="s")
    fn = functools.partial(
        pl.kernel,
        mesh=mesh,
        compiler_params=pltpu.CompilerParams(
            needs_layout_passes=False, use_tc_tiling_on_sc=True),
        out_type=jax.ShapeDtypeStruct((NW * N_GLOBAL * TW,), jnp.float32),
        scratch_types=[
            pltpu.VMEM((CT,), jnp.int32),
            pltpu.VMEM((CT + 16,), jnp.int32),
            pltpu.VMEM((N_GLOBAL * 16,), jnp.float32),
            pltpu.VMEM((N_GLOBAL * TW,), jnp.float32),
            pltpu.VMEM((NBUF, 16, DG), jnp.float32),
            pltpu.VMEM((NBUF, 16), jnp.int32),
            pltpu.VMEM((NBUF, 128), jnp.int32),
            pltpu.VMEM((NBUF, 128), jnp.float32),
            pltpu.VMEM((16, 16), jnp.float32),
            pltpu.SemaphoreType.DMA((NBUF,)),
            pltpu.SemaphoreType.DMA((NBUF,)),
        ],
    )(_edge_body)
    return fn(src_pad, dst_pad, T, as8, adstb)


def _projf_body(xf3_ref, wf_ref, bfp_ref, out_ref):
    xfb = xf3_ref[...].reshape(N_GLOBAL, IN_F)
    out_ref[...] = (
        jnp.dot(xfb, wf_ref[...], preferred_element_type=jnp.float32)
        + bfp_ref[...]).reshape(1, N_GLOBAL, LLM)


def _run_projf(xf3, Wf, bf):
    nj = N_FINE // N_GLOBAL + 1
    return pl.pallas_call(
        _projf_body,
        grid=(nj - 1,),
        in_specs=[
            pl.BlockSpec((N_GLOBAL, 1, 1, IN_F), lambda j: (0, j, 0, 0)),
            pl.BlockSpec((IN_F, LLM), lambda j: (0, 0)),
            pl.BlockSpec((1, LLM), lambda j: (0, 0)),
        ],
        out_specs=pl.BlockSpec((1, N_GLOBAL, LLM), lambda j: (j + 1, 0, 0)),
        out_shape=jax.ShapeDtypeStruct((nj, N_GLOBAL, LLM), jnp.float32),
    )(xf3, Wf, bf)


def _epi_body(full_ref, parts_ref, tg_ref, asl_ref, adstb_ref,
              xgln_ref, bias_ref, r_ref, wg_ref, bgp_ref, out_ref):
    p = jnp.sum(parts_ref[...], axis=0)
    snum = p[:, :DG]
    den = p[:, DG:DG + HEADS]
    tg = tg_ref[...]
    asl = asl_ref[...]
    ad = adstb_ref[:, :HEADS]
    bub = adstb_ref[:, HEADS:]
    sm = asl + ad
    al = jnp.maximum(sm, 0.2 * sm)
    exl = jnp.exp(al - bub)
    rmat = r_ref[...]
    exrep = jnp.dot(exl, rmat, preferred_element_type=jnp.float32)
    snum = snum + exrep * tg
    drep = (jnp.dot(den + exl, rmat, preferred_element_type=jnp.float32)
            + 1e-16)
    xgn = snum / drep + bias_ref[...] + xgln_ref[...]
    out_ref[...] = (
        jnp.dot(xgn, wg_ref[...], preferred_element_type=jnp.float32)
        + bgp_ref[...]).reshape(1, N_GLOBAL, LLM)


def _run_epi(full, parts, Tg, asl, adstb, xgln, bias, R, Wg, bg):
    nj = N_FINE // N_GLOBAL + 1
    return pl.pallas_call(
        _epi_body,
        grid=(1,),
        in_specs=[
            pl.BlockSpec((1, N_GLOBAL, LLM), lambda j: (0, 0, 0)),
            pl.BlockSpec((NW, N_GLOBAL, TW), lambda j: (0, 0, 0)),
            pl.BlockSpec((N_GLOBAL, DG), lambda j: (0, 0)),
            pl.BlockSpec((N_GLOBAL, HEADS), lambda j: (0, 0)),
            pl.BlockSpec((N_GLOBAL, 16), lambda j: (0, 0)),
            pl.BlockSpec((N_GLOBAL, DG), lambda j: (0, 0)),
            pl.BlockSpec((1, DG), lambda j: (0, 0)),
            pl.BlockSpec((HEADS, DG), lambda j: (0, 0)),
            pl.BlockSpec((IN_F, LLM), lambda j: (0, 0)),
            pl.BlockSpec((1, LLM), lambda j: (0, 0)),
        ],
        out_specs=pl.BlockSpec((1, N_GLOBAL, LLM), lambda j: (0, 0, 0)),
        out_shape=jax.ShapeDtypeStruct((nj, N_GLOBAL, LLM), jnp.float32),
        input_output_aliases={0: 0},
    )(full, parts, Tg, asl, adstb, xgln, bias, R, Wg, bg)


def kernel(x_fine, x_global, edge_index, gamma_f, beta_f, W1, b1, W2, b2,
           gamma_g, beta_g, W_src, W_dst, att_src, att_dst, bias_gat,
           Wg, bg, Wf, bf):
    f32 = jnp.float32
    eye = jnp.eye(HEADS, dtype=f32)
    A = (att_src[:, :, None] * eye[:, None, :]).reshape(DG, HEADS)
    Ad = (att_dst[:, :, None] * eye[:, None, :]).reshape(DG, HEADS)
    R = jnp.broadcast_to(eye[:, :, None], (HEADS, HEADS, ATT)).reshape(HEADS, DG)

    src = edge_index[0]
    dst = edge_index[1]
    padn = EPAD - E
    src_pad = jnp.concatenate([src, jnp.zeros((padn,), src.dtype)])
    dst_pad = jnp.concatenate([dst, jnp.zeros((padn,), dst.dtype)])

    T, asrc, xf, mx = _run_pre(
        x_fine, gamma_f.reshape(1, IN_F), beta_f.reshape(1, IN_F),
        W1, b1.reshape(1, HID), W2, b2.reshape(1, DG), W_src, A)
    xgln, adstb = _run_glob(
        x_global, gamma_g.reshape(1, DG), beta_g.reshape(1, DG), W_dst, Ad, mx)
    parts = _run_edges(src_pad, dst_pad, T, asrc.reshape(N_FINE * HEADS),
                       adstb.reshape(N_GLOBAL * 16))
    parts = parts.reshape(NW, N_GLOBAL, TW)
    Tg = lax.slice(T, (0, 0), (N_GLOBAL, DG))
    asl = lax.slice(asrc, (0, 0), (N_GLOBAL, HEADS))
    xf3 = xf.reshape(N_GLOBAL, N_FINE // N_GLOBAL, 1, IN_F)
    full = _run_projf(xf3, Wf, bf.reshape(1, LLM))
    full = _run_epi(full, parts, Tg, asl, adstb, xgln,
                    bias_gat.reshape(1, DG), R, Wg, bg.reshape(1, LLM))
    return jnp.swapaxes(full, 0, 1)
